# Initial kernel scaffold; baseline (speedup 1.0000x reference)
import jax
import jax.numpy as jnp
from jax.experimental import pallas as pl


def kernel(grid, guide, image):
    raise NotImplementedError("write your pallas kernel here")


# speedup vs baseline: 887.1958x; 887.1958x over previous
import jax
import jax.numpy as jnp
from jax.experimental import pallas as pl
from jax.experimental.pallas import tpu as pltpu

_B, _H, _W = 4, 512, 512
_GD, _GH, _GW = 8, 16, 16
_C12 = 12
_ROWS = 16
_NS = _H // _ROWS


def _strip_j0(s):
    return jnp.clip((s - 1) // 2, 0, _GH - 2)


def _bsa_kernel(g0_ref, g1_ref, guide_ref, img_ref, out_ref):
    s = pl.program_id(1)
    f32 = jnp.float32

    i_idx = jax.lax.broadcasted_iota(jnp.int32, (_GW, _W), 0).astype(f32)
    x_idx = jax.lax.broadcasted_iota(jnp.int32, (_GW, _W), 1).astype(f32)
    gx = jnp.clip((x_idx + 0.5) * (_GW / _W), 0.5, _GW - 0.5)
    xwt = jnp.maximum(1.0 - jnp.abs(i_idx + 0.5 - gx), 0.0)
    dn = (((1,), (0,)), ((), ()))
    u0 = jax.lax.dot_general(g0_ref[0, 0], xwt, dn,
                             precision=jax.lax.Precision.HIGHEST,
                             preferred_element_type=f32)
    u1 = jax.lax.dot_general(g1_ref[0, 0], xwt, dn,
                             precision=jax.lax.Precision.HIGHEST,
                             preferred_element_type=f32)

    r = jax.lax.broadcasted_iota(jnp.int32, (_ROWS, 1), 0).astype(f32)
    y = s.astype(f32) * _ROWS + r
    gy = jnp.clip((y + 0.5) * (_GH / _H), 0.5, _GH - 0.5)
    fy = jnp.clip(jnp.floor(gy - 0.5), 0.0, _GH - 2.0)
    wy1 = gy - 0.5 - fy
    wy0 = 1.0 - wy1

    gz = jnp.clip(guide_ref[0] * _GD, 0.5, _GD - 0.5)

    acc = [None] * _C12
    for z in range(_GD):
        zw = jnp.maximum(1.0 - jnp.abs(z + 0.5 - gz), 0.0)
        a0 = zw * wy0
        a1 = zw * wy1
        for c in range(_C12):
            row = z * _C12 + c
            t = a0 * u0[row] + a1 * u1[row]
            acc[c] = t if acc[c] is None else acc[c] + t

    img = img_ref[0]
    for o in range(3):
        res = (acc[4 * o + 0] * img[0] + acc[4 * o + 1] * img[1]
               + acc[4 * o + 2] * img[2] + acc[4 * o + 3])
        out_ref[0, o] = res


@jax.jit
def kernel(grid, guide, image):
    B, C12, gd, gh, gw = grid.shape
    g3 = jnp.transpose(grid, (0, 3, 2, 1, 4)).reshape(B, gh, gd * C12, gw)

    out = pl.pallas_call(
        _bsa_kernel,
        grid=(_B, _NS),
        in_specs=[
            pl.BlockSpec((1, 1, gd * C12, gw), lambda b, s: (b, _strip_j0(s), 0, 0)),
            pl.BlockSpec((1, 1, gd * C12, gw), lambda b, s: (b, _strip_j0(s) + 1, 0, 0)),
            pl.BlockSpec((1, _ROWS, _W), lambda b, s: (b, s, 0)),
            pl.BlockSpec((1, 3, _ROWS, _W), lambda b, s: (b, 0, s, 0)),
        ],
        out_specs=pl.BlockSpec((1, 3, _ROWS, _W), lambda b, s: (b, 0, s, 0)),
        out_shape=jax.ShapeDtypeStruct((B, 3, _H, _W), jnp.float32),
        compiler_params=pltpu.CompilerParams(
            dimension_semantics=("parallel", "parallel")),
    )(g3, g3, guide, image)
    return out
